# Initial kernel scaffold; baseline (speedup 1.0000x reference)
import jax
import jax.numpy as jnp
from jax.experimental import pallas as pl


def kernel(x, ln_w, ln_b, w1, b1, w2, b2):
    raise NotImplementedError("write your pallas kernel here")


# speedup vs baseline: 15.4696x; 15.4696x over previous
import numpy as np
import jax
import jax.numpy as jnp
from jax import lax
from jax.experimental import pallas as pl
from jax.experimental.pallas import tpu as pltpu

H = W = 48
C = 96
HW = H * W
HALF = 6
PAD = HALF * W
MAXC = 16
GR = 8
GS = GR * W
GB = (GR + 2 * HALF) * W
NG = H // GR
HI = lax.Precision.HIGHEST


def _resize_mat(in_size, out_size):
    inv = in_size / out_size
    ks = max(inv, 1.0)
    sample_f = (np.arange(out_size) + 0.5) * inv - 0.5
    t = np.abs(sample_f[None, :] - np.arange(in_size)[:, None]) / ks
    w = np.maximum(0.0, 1.0 - t)
    w = w / np.sum(w, axis=0, keepdims=True)
    return w.T.astype(np.float32)


_D_MAT = _resize_mat(H, H // 2)
_U_MAT = _resize_mat(H // 2, H)


def _body(xpad_ref, xt2_ref, xt_ref, d_ref, u_ref, lnw_ref, lnb_ref,
          w1_ref, b1_ref, w2_ref, b2_ref, out_ref, pf_ref):
    f32 = jnp.float32
    dmat = d_ref[...]
    umat = u_ref[...]

    def df_body(c, acc):
        xct = xt_ref[pl.ds(c * W, W), :]
        t1 = lax.dot_general(xct, dmat, (((1,), (1,)), ((), ())),
                             preferred_element_type=f32, precision=HI)
        t2 = lax.dot_general(dmat, t1, (((1,), (0,)), ((), ())),
                             preferred_element_type=f32, precision=HI)
        t3 = lax.dot_general(t2, umat, (((1,), (1,)), ((), ())),
                             preferred_element_type=f32, precision=HI)
        up = lax.dot_general(umat, t3, (((1,), (0,)), ((), ())),
                             preferred_element_type=f32, precision=HI)
        return acc + jnp.abs(xct - up)

    dft = lax.fori_loop(0, C, df_body, jnp.zeros((W, H), f32))
    mn = jnp.min(dft)
    mx = jnp.max(dft)
    kct = 1.0 + jnp.round((dft - mn) / (mx - mn + 1e-8) * (MAXC - 1))
    kct8 = jnp.concatenate([kct] * GR, axis=0)

    xpad = xpad_ref[...]
    nrm = jnp.sqrt(jnp.sum(xpad * xpad, axis=1, keepdims=True))
    pf_ref[...] = xpad / jnp.maximum(nrm, 1e-12)
    xt2 = xt2_ref[...]
    nrmt = jnp.maximum(jnp.sqrt(jnp.sum(xt2 * xt2, axis=0, keepdims=True)),
                       1e-12)

    lnw = lnw_ref[...]
    lnb = lnb_ref[...]
    w1 = w1_ref[...]
    b1 = b1_ref[...]
    w2 = w2_ref[...]
    b2 = b2_ref[...]

    ss = lax.broadcasted_iota(jnp.int32, (GS, GB), 0)
    jj = lax.broadcasted_iota(jnp.int32, (GS, GB), 1)
    gi = ss // W
    px = ss - gi * W
    bj = jj // W
    cx = jj - bj * W
    colmask = jnp.abs(cx - px) <= HALF
    dyrel = bj - gi
    winmask = colmask & (dyrel >= 0) & (dyrel <= 2 * HALF)
    hh48 = lax.broadcasted_iota(jnp.int32, (GS, H), 1)

    for g in range(NG):
        y0 = g * GR
        kc = jnp.sum(jnp.where(hh48 == (y0 + gi[:, :H]), kct8, 0.0),
                     axis=1, keepdims=True)

        rows_pf = pf_ref[pl.ds((y0 + HALF) * W, GS), :]
        band_u = xpad_ref[pl.ds(y0 * W, GB), :]
        sraw = lax.dot_general(rows_pf, band_u, (((1,), (1,)), ((), ())),
                               preferred_element_type=f32)
        bnrm = lax.slice(nrmt, (0, y0 * W), (1, y0 * W + GB))
        s = sraw / bnrm
        rowvalid = (y0 + bj - HALF >= 0) & (y0 + bj - HALF < H)
        sm = jnp.where(winmask & rowvalid, s, -1e9)

        def topk_body(r, carry):
            work, thr = carry
            cur = jnp.max(work, axis=1, keepdims=True)
            thr = jnp.where(kc == lax.convert_element_type(r + 1, f32), cur, thr)
            work = jnp.where(work >= cur, -jnp.inf, work)
            return work, thr

        _, thr = lax.fori_loop(0, MAXC, topk_body,
                               (sm, jnp.zeros((GS, 1), f32)))

        w_un = jnp.where(sm >= thr, jnp.exp(sm), 0.0)
        z = jnp.sum(w_un, axis=1, keepdims=True)
        wn = w_un / z

        agg = lax.dot_general(wn, band_u, (((1,), (0,)), ((), ())),
                              preferred_element_type=f32, precision=HI)

        xrow_u = xpad_ref[pl.ds((y0 + HALF) * W, GS), :]
        mu = jnp.mean(xrow_u, axis=1, keepdims=True)
        ctr = xrow_u - mu
        var = jnp.mean(ctr * ctr, axis=1, keepdims=True)
        xln = ctr / jnp.sqrt(var + 1e-5) * lnw + lnb

        enh = agg + xln
        hmid = lax.dot_general(enh, w1, (((1,), (1,)), ((), ())),
                               preferred_element_type=f32) + b1
        hmid = jnp.maximum(hmid, 0.0)
        ffn = lax.dot_general(hmid, w2, (((1,), (1,)), ((), ())),
                              preferred_element_type=f32) + b2
        out_ref[pl.ds(y0 * W, GS), :] = enh + ffn


def kernel(x, ln_w, ln_b, w1, b1, w2, b2):
    xf = x[0].transpose(1, 2, 0).reshape(HW, C)
    xpad = jnp.pad(xf, ((PAD, PAD), (0, 0)))
    xt2 = xpad.T
    xt = x[0].transpose(0, 2, 1).reshape(C * W, H)

    out = pl.pallas_call(
        _body,
        out_shape=jax.ShapeDtypeStruct((HW, C), jnp.float32),
        scratch_shapes=[pltpu.VMEM((HW + 2 * PAD, C), jnp.float32)],
    )(xpad, xt2, xt, jnp.asarray(_D_MAT), jnp.asarray(_U_MAT),
      ln_w.reshape(1, C), ln_b.reshape(1, C),
      w1, b1.reshape(1, 2 * C), w2, b2.reshape(1, C))

    return out.reshape(H, W, C).transpose(2, 0, 1)[None]


# speedup vs baseline: 25.0943x; 1.6222x over previous
import numpy as np
import jax
import jax.numpy as jnp
from jax import lax
from jax.experimental import pallas as pl
from jax.experimental.pallas import tpu as pltpu

H = W = 48
C = 96
HW = H * W
HALF = 6
PAD = HALF * W
MAXC = 16
GR = 8
GS = GR * W
GB = (GR + 2 * HALF) * W
NG = H // GR
HI = lax.Precision.HIGHEST


def _resize_mat(in_size, out_size):
    inv = in_size / out_size
    ks = max(inv, 1.0)
    sample_f = (np.arange(out_size) + 0.5) * inv - 0.5
    t = np.abs(sample_f[None, :] - np.arange(in_size)[:, None]) / ks
    w = np.maximum(0.0, 1.0 - t)
    w = w / np.sum(w, axis=0, keepdims=True)
    return w.T.astype(np.float32)


_D_MAT = _resize_mat(H, H // 2)
_U_MAT = _resize_mat(H // 2, H)

_M_W = (_U_MAT.astype(np.float64) @ _D_MAT.astype(np.float64))
_DBAND = 3


def _shift_coefs():
    cf = np.zeros((C * W, 2 * _DBAND + 1), np.float32)
    for wp in range(W):
        for j, d in enumerate(range(-_DBAND, _DBAND + 1)):
            if 0 <= wp + d < W:
                cf[wp::W, j] = np.float32(_M_W[wp, wp + d])
    return cf


_CF = _shift_coefs()


def _body(xpad_ref, xt2_ref, xt_ref, d_ref, u_ref, cf_ref, lnw_ref, lnb_ref,
          w1_ref, b1_ref, w2_ref, b2_ref, out_ref, pf_ref):
    f32 = jnp.float32
    dmat = d_ref[...]
    umat = u_ref[...]

    xt = xt_ref[...]
    p1 = lax.dot_general(xt, dmat, (((1,), (1,)), ((), ())),
                         preferred_element_type=f32, precision=HI)
    p2 = lax.dot_general(p1, umat, (((1,), (1,)), ((), ())),
                         preferred_element_type=f32, precision=HI)
    p2p = jnp.pad(p2, ((_DBAND, _DBAND), (0, 0)))
    cf = cf_ref[...]
    up = cf[:, 0:1] * lax.slice(p2p, (0, 0), (C * W, H))
    for j in range(1, 2 * _DBAND + 1):
        up = up + cf[:, j:j + 1] * lax.slice(p2p, (j, 0), (j + C * W, H))
    dabs = jnp.abs(xt - up)
    for rows in (2304, 1152, 576, 288, 144):
        dabs = lax.slice(dabs, (0, 0), (rows, H)) + \
            lax.slice(dabs, (rows, 0), (2 * rows, H))
    dft = (lax.slice(dabs, (0, 0), (W, H)) +
           lax.slice(dabs, (W, 0), (2 * W, H)) +
           lax.slice(dabs, (2 * W, 0), (3 * W, H)))
    mn = jnp.min(dft)
    mx = jnp.max(dft)
    kct = 1.0 + jnp.round((dft - mn) / (mx - mn + 1e-8) * (MAXC - 1))
    kct8 = jnp.concatenate([kct] * GR, axis=0)

    xpad = xpad_ref[...]
    nrm = jnp.sqrt(jnp.sum(xpad * xpad, axis=1, keepdims=True))
    pf_ref[...] = xpad / jnp.maximum(nrm, 1e-12)
    xt2 = xt2_ref[...]
    nrmt = jnp.maximum(jnp.sqrt(jnp.sum(xt2 * xt2, axis=0, keepdims=True)),
                       1e-12)

    lnw = lnw_ref[...]
    lnb = lnb_ref[...]
    w1 = w1_ref[...]
    b1 = b1_ref[...]
    w2 = w2_ref[...]
    b2 = b2_ref[...]

    ss = lax.broadcasted_iota(jnp.int32, (GS, GB), 0)
    jj = lax.broadcasted_iota(jnp.int32, (GS, GB), 1)
    gi = ss // W
    px = ss - gi * W
    bj = jj // W
    cx = jj - bj * W
    colmask = jnp.abs(cx - px) <= HALF
    dyrel = bj - gi
    winmask = colmask & (dyrel >= 0) & (dyrel <= 2 * HALF)
    hh48 = lax.broadcasted_iota(jnp.int32, (GS, H), 1)

    for g in range(NG):
        y0 = g * GR
        kc = jnp.sum(jnp.where(hh48 == (y0 + gi[:, :H]), kct8, 0.0),
                     axis=1, keepdims=True)

        rows_pf = pf_ref[pl.ds((y0 + HALF) * W, GS), :]
        band_u = xpad_ref[pl.ds(y0 * W, GB), :]
        sraw = lax.dot_general(rows_pf, band_u, (((1,), (1,)), ((), ())),
                               preferred_element_type=f32)
        bnrm = lax.slice(nrmt, (0, y0 * W), (1, y0 * W + GB))
        s = sraw / bnrm
        rowvalid = (y0 + bj - HALF >= 0) & (y0 + bj - HALF < H)
        sm = jnp.where(winmask & rowvalid, s, -1e9)

        def topk_body(r, carry):
            work, thr = carry
            cur = jnp.max(work, axis=1, keepdims=True)
            thr = jnp.where(kc == lax.convert_element_type(r + 1, f32), cur, thr)
            work = jnp.where(work >= cur, -jnp.inf, work)
            return work, thr

        _, thr = lax.fori_loop(0, MAXC, topk_body,
                               (sm, jnp.zeros((GS, 1), f32)))

        w_un = jnp.where(sm >= thr, jnp.exp(sm), 0.0)
        z = jnp.sum(w_un, axis=1, keepdims=True)
        wn = w_un / z

        agg = lax.dot_general(wn, band_u, (((1,), (0,)), ((), ())),
                              preferred_element_type=f32, precision=HI)

        xrow_u = xpad_ref[pl.ds((y0 + HALF) * W, GS), :]
        mu = jnp.mean(xrow_u, axis=1, keepdims=True)
        ctr = xrow_u - mu
        var = jnp.mean(ctr * ctr, axis=1, keepdims=True)
        xln = ctr / jnp.sqrt(var + 1e-5) * lnw + lnb

        enh = agg + xln
        hmid = lax.dot_general(enh, w1, (((1,), (1,)), ((), ())),
                               preferred_element_type=f32) + b1
        hmid = jnp.maximum(hmid, 0.0)
        ffn = lax.dot_general(hmid, w2, (((1,), (1,)), ((), ())),
                              preferred_element_type=f32) + b2
        out_ref[pl.ds(y0 * W, GS), :] = enh + ffn


def kernel(x, ln_w, ln_b, w1, b1, w2, b2):
    xf = x[0].transpose(1, 2, 0).reshape(HW, C)
    xpad = jnp.pad(xf, ((PAD, PAD), (0, 0)))
    xt2 = xpad.T
    xt = x[0].transpose(0, 2, 1).reshape(C * W, H)

    out = pl.pallas_call(
        _body,
        out_shape=jax.ShapeDtypeStruct((HW, C), jnp.float32),
        scratch_shapes=[pltpu.VMEM((HW + 2 * PAD, C), jnp.float32)],
    )(xpad, xt2, xt, jnp.asarray(_D_MAT), jnp.asarray(_U_MAT),
      jnp.asarray(_CF), ln_w.reshape(1, C), ln_b.reshape(1, C),
      w1, b1.reshape(1, 2 * C), w2, b2.reshape(1, C))

    return out.reshape(H, W, C).transpose(2, 0, 1)[None]
